# baseline (speedup 1.0000x reference)
import functools
import jax
import jax.numpy as jnp
from jax import lax
from jax.experimental import pallas as pl
from jax.experimental.pallas import tpu as pltpu
from jax.experimental.pallas import tpu_sc as plsc

_B = 16384
_D = 64
_NW = 32
_BPW = _B // _NW
_CHUNK = 128
_NCH = _BPW // _CHUNK


def _make_kernel():
    mesh = plsc.VectorSubcoreMesh(core_axis_name="c", subcore_axis_name="s")

    @functools.partial(
        pl.kernel,
        mesh=mesh,
        out_type=jax.ShapeDtypeStruct((_NW, 128), jnp.float32),
        scratch_types=[
            pltpu.VMEM((_NCH, _CHUNK), jnp.int32),
            pltpu.VMEM((_BPW * _D,), jnp.float32),
            pltpu.VMEM((_BPW, 128), jnp.float32),
            pltpu.VMEM((_BPW * 16,), jnp.float32),
            pltpu.VMEM((128,), jnp.float32),
            pltpu.SemaphoreType.DMA,
        ],
    )
    def k(feat_hbm, rowidx_hbm, pm_hbm, table_hbm, out_hbm,
          idx_v, feat_v, rows_v, pm_v, acc_v, sem):
        wid = lax.axis_index("s") * 2 + lax.axis_index("c")
        pltpu.sync_copy(rowidx_hbm.at[pl.ds(wid * _NCH, _NCH)], idx_v)
        copies = []
        for j in range(_NCH):
            copies.append(
                pltpu.async_copy(
                    table_hbm.at[idx_v.at[j]],
                    rows_v.at[pl.ds(j * _CHUNK, _CHUNK)],
                    sem,
                )
            )
        pltpu.sync_copy(feat_hbm.at[pl.ds(wid * _BPW * _D, _BPW * _D)], feat_v)
        pltpu.sync_copy(pm_hbm.at[pl.ds(wid * _BPW * 16, _BPW * 16)], pm_v)
        for c in copies:
            c.wait()

        zero = jnp.zeros((16,), jnp.float32)

        def body(i, accs):
            a0, a1, a2, a3 = accs
            m = pm_v[pl.ds(i * 16, 16)]
            fb = i * _D
            out = []
            for kk, a in enumerate((a0, a1, a2, a3)):
                f = feat_v[pl.ds(fb + kk * 16, 16)]
                dlo = f - rows_v[i, pl.ds(kk * 16, 16)]
                dhi = f - rows_v[i, pl.ds(64 + kk * 16, 16)]
                slo = dlo * dlo
                shi = dhi * dhi
                out.append(a + shi + (slo - shi) * m)
            return tuple(out)

        a0, a1, a2, a3 = lax.fori_loop(0, _BPW, body, (zero, zero, zero, zero))
        acc_v[pl.ds(0, 16)] = a0 + a1
        acc_v[pl.ds(16, 16)] = a2 + a3
        for j in range(2, 8):
            acc_v[pl.ds(j * 16, 16)] = zero
        pltpu.sync_copy(acc_v, out_hbm.at[wid])

    return k


_sc_kernel = _make_kernel()


def kernel(feat, labels, centers):
    labels = labels.astype(jnp.int32)
    table = centers.reshape(500000, 128)
    rowidx = (labels >> 1).reshape(_NW * _NCH, _CHUNK)
    par_even = (1 - (labels & 1)).astype(jnp.float32)
    pm = jnp.broadcast_to(par_even[:, None], (_B, 16)).reshape(-1)
    feat1d = feat.reshape(-1)
    partials = _sc_kernel(feat1d, rowidx, pm, table)
    return jnp.sum(partials) * (0.5 / _B)


# speedup vs baseline: 1.1253x; 1.1253x over previous
import functools
import jax
import jax.numpy as jnp
from jax import lax
from jax.experimental import pallas as pl
from jax.experimental.pallas import tpu as pltpu
from jax.experimental.pallas import tpu_sc as plsc

_B = 16384
_D = 64
_NW = 32
_BPW = _B // _NW
_CHUNK = 128
_NCH = _BPW // _CHUNK


def _make_kernel():
    mesh = plsc.VectorSubcoreMesh(core_axis_name="c", subcore_axis_name="s")

    @functools.partial(
        pl.kernel,
        mesh=mesh,
        out_type=jax.ShapeDtypeStruct((_NW, 128), jnp.float32),
        scratch_types=[
            pltpu.VMEM((_NCH, _CHUNK), jnp.int32),
            pltpu.VMEM((_BPW * _D,), jnp.float32),
            pltpu.VMEM((_BPW, 128), jnp.float32),
            pltpu.VMEM((128,), jnp.float32),
            pltpu.SemaphoreType.DMA,
        ],
    )
    def k(feat_hbm, idx_hbm, table_hbm, out_hbm,
          idx_v, feat_v, rows_v, acc_v, sem):
        wid = lax.axis_index("s") * 2 + lax.axis_index("c")
        pltpu.sync_copy(idx_hbm.at[pl.ds(wid * _NCH, _NCH)], idx_v)
        copies = []
        for j in range(_NCH):
            copies.append(
                pltpu.async_copy(
                    table_hbm.at[idx_v.at[j]],
                    rows_v.at[pl.ds(j * _CHUNK, _CHUNK)],
                    sem,
                )
            )
        pltpu.sync_copy(feat_hbm.at[pl.ds(wid * _BPW * _D, _BPW * _D)], feat_v)
        for c in copies:
            c.wait()

        zero = jnp.zeros((16,), jnp.float32)

        def body(i, accs):
            accs4 = list(accs)
            fb = i * _D
            for kk in range(4):
                d = (feat_v[pl.ds(fb + kk * 16, 16)]
                     - rows_v[i, pl.ds(kk * 16, 16)])
                accs4[kk] = accs4[kk] + d * d
            return tuple(accs4)

        a0, a1, a2, a3 = lax.fori_loop(0, _BPW, body, (zero, zero, zero, zero))
        acc_v[pl.ds(0, 16)] = (a0 + a1) + (a2 + a3)
        for j in range(1, 8):
            acc_v[pl.ds(j * 16, 16)] = zero
        pltpu.sync_copy(acc_v, out_hbm.at[wid])

    return k


_sc_kernel = _make_kernel()


def kernel(feat, labels, centers):
    labels = labels.astype(jnp.int32)
    table = jnp.pad(centers, ((0, 0), (0, 64)))
    rowidx = labels.reshape(_NW * _NCH, _CHUNK)
    feat1d = feat.reshape(-1)
    partials = _sc_kernel(feat1d, rowidx, table)
    return jnp.sum(partials) * (0.5 / _B)


# speedup vs baseline: 1.9057x; 1.6935x over previous
import functools
import jax
import jax.numpy as jnp
from jax import lax
from jax.experimental import pallas as pl
from jax.experimental.pallas import tpu as pltpu
from jax.experimental.pallas import tpu_sc as plsc

_B = 16384
_D = 64
_NW = 32
_BPW = _B // _NW
_NBUF = 8


def _make_kernel():
    mesh = plsc.VectorSubcoreMesh(core_axis_name="c", subcore_axis_name="s")

    @functools.partial(
        pl.kernel,
        mesh=mesh,
        out_type=jax.ShapeDtypeStruct((_NW, 128), jnp.float32),
        compiler_params=pltpu.CompilerParams(needs_layout_passes=False),
        scratch_types=[
            pltpu.VMEM((_BPW,), jnp.int32),
            pltpu.VMEM((_BPW,), jnp.int32),
            pltpu.VMEM((_BPW,), jnp.int32),
            pltpu.VMEM((_BPW * _D,), jnp.float32),
            pltpu.VMEM((_NBUF, _D, 128), jnp.float32),
            pltpu.VMEM((128,), jnp.float32),
            pltpu.SemaphoreType.DMA((_NBUF,)),
        ],
    )
    def k(feat_hbm, pa_hbm, pe1_hbm, pe2_hbm, tableT_hbm, out_hbm,
          pa_v, pe1_v, pe2_v, feat_v, blk_v, acc_v, sems):
        wid = lax.axis_index("s") * 2 + lax.axis_index("c")
        base = wid * _BPW
        pltpu.sync_copy(pa_hbm.at[pl.ds(base, _BPW)], pa_v)
        pltpu.sync_copy(pe1_hbm.at[pl.ds(base, _BPW)], pe1_v)
        pltpu.sync_copy(pe2_hbm.at[pl.ds(base, _BPW)], pe2_v)
        pltpu.sync_copy(feat_hbm.at[pl.ds(base * _D, _BPW * _D)], feat_v)

        lane = lax.iota(jnp.int32, 16)
        zero = jnp.zeros((16,), jnp.float32)

        def extract(ref, i):
            v16 = ref[pl.ds((i >> 4) * 16, 16)]
            return jnp.sum(jnp.where(lane == (i & 15), v16, 0))

        def enqueue(e):
            cb = (e & 0x3FFF) - 1
            slot = e >> 14
            pltpu.async_copy(
                tableT_hbm.at[:, pl.ds(pl.multiple_of(cb * 128, 128), 128)],
                blk_v.at[slot],
                sems.at[slot],
            )

        def body(i, accs):
            a = extract(pa_v, i)
            e1 = extract(pe1_v, i)
            e2 = extract(pe2_v, i)
            y = a & 0xFFFFF
            nb = (a >> 20) & 1
            slu = (a >> 21) & 7

            @pl.when(e1 > 0)
            def _():
                enqueue(e1)

            @pl.when(e2 > 0)
            def _():
                enqueue(e2)

            @pl.when(nb == 1)
            def _():
                pltpu.make_async_copy(
                    tableT_hbm.at[:, pl.ds(0, 128)],
                    blk_v.at[slu],
                    sems.at[slu],
                ).wait()

            o16 = jnp.broadcast_to(y & 127, (16,)) + lane * 0
            fb = i * _D
            accs4 = list(accs)
            for kk in range(4):
                jvec = kk * 16 + lane
                c = plsc.load_gather(blk_v.at[slu], [jvec, o16])
                d = feat_v[pl.ds(fb + kk * 16, 16)] - c
                accs4[kk] = accs4[kk] + d * d
            return tuple(accs4)

        a0, a1, a2, a3 = lax.fori_loop(0, _BPW, body, (zero, zero, zero, zero))
        acc_v[pl.ds(0, 16)] = (a0 + a1) + (a2 + a3)
        for j in range(1, 8):
            acc_v[pl.ds(j * 16, 16)] = zero
        pltpu.sync_copy(acc_v, out_hbm.at[wid])

    return k


_sc_kernel = _make_kernel()


def _schedule(y_sorted_2d):
    cb = y_sorted_2d >> 7
    prev = jnp.concatenate([cb[:, :1] - 1, cb[:, :-1]], axis=1)
    nb = (cb != prev).astype(jnp.int32)
    bidx = jnp.cumsum(nb, axis=1) - 1
    slot = bidx % _NBUF
    pa = y_sorted_2d + (nb << 20) + (slot << 21)

    nblk = jnp.sum(nb, axis=1, keepdims=True)
    pos = jnp.broadcast_to(jnp.arange(_BPW, dtype=jnp.int32), cb.shape)
    woff = jnp.arange(_NW, dtype=jnp.int32)[:, None] * _BPW
    seg = (bidx + woff).reshape(-1)
    firstpos = jax.ops.segment_min(pos.reshape(-1), seg,
                                   num_segments=_NW * _BPW).reshape(_NW, _BPW)
    blkcb = jax.ops.segment_min(cb.reshape(-1), seg,
                                num_segments=_NW * _BPW).reshape(_NW, _BPW)
    bvalid = jnp.broadcast_to(jnp.arange(_BPW, dtype=jnp.int32),
                              cb.shape) < nblk
    pe1 = jnp.where(
        (jnp.arange(_BPW) < _NBUF - 1) & bvalid,
        (blkcb + 1) + ((jnp.arange(_BPW, dtype=jnp.int32) % _NBUF) << 14),
        0,
    )
    nxt = jnp.roll(blkcb, -(_NBUF - 1), axis=1)
    nxt_valid = (jnp.arange(_BPW) + _NBUF - 1 < nblk) & bvalid
    nxt_slot = (jnp.arange(_BPW, dtype=jnp.int32) + _NBUF - 1) % _NBUF
    val = jnp.where(nxt_valid, (nxt + 1) + (nxt_slot << 14), 0)
    pe2 = jnp.zeros_like(pa).reshape(-1)
    tgt = (firstpos + woff).reshape(-1)
    pe2 = pe2.at[jnp.where(bvalid.reshape(-1), tgt, _NW * _BPW - 1)].max(
        jnp.where(bvalid.reshape(-1), val.reshape(-1), 0))
    return pa, pe1, pe2.reshape(_NW, _BPW)


def kernel(feat, labels, centers):
    y = labels.astype(jnp.int32)
    order = jnp.argsort(y)
    ys = y[order].reshape(_NW, _BPW)
    featp = feat[order]
    pa, pe1, pe2 = _schedule(ys)
    partials = _sc_kernel(
        featp.reshape(-1), pa.reshape(-1), pe1.reshape(-1), pe2.reshape(-1),
        centers.T)
    return jnp.sum(partials) * (0.5 / _B)


# speedup vs baseline: 3.6755x; 1.9287x over previous
import functools
import jax
import jax.numpy as jnp
from jax import lax
from jax.experimental import pallas as pl
from jax.experimental.pallas import tpu as pltpu
from jax.experimental.pallas import tpu_sc as plsc

_B = 16384
_D = 64
_NW = 32
_BPW = _B // _NW
_NBUF = 8


def _make_kernel():
    mesh = plsc.VectorSubcoreMesh(core_axis_name="c", subcore_axis_name="s")

    @functools.partial(
        pl.kernel,
        mesh=mesh,
        out_type=jax.ShapeDtypeStruct((_NW, 128), jnp.float32),
        compiler_params=pltpu.CompilerParams(needs_layout_passes=False),
        scratch_types=[
            pltpu.VMEM((_BPW,), jnp.int32),
            pltpu.VMEM((_BPW,), jnp.int32),
            pltpu.VMEM((_BPW * 16,), jnp.int32),
            pltpu.VMEM((_BPW * _D,), jnp.float32),
            pltpu.VMEM((_NBUF, _D, 128), jnp.float32),
            pltpu.VMEM((128,), jnp.float32),
            pltpu.SemaphoreType.DMA((_NBUF,)),
        ],
    )
    def k(feat_hbm, ct_hbm, pe1_hbm, oi_hbm, tableT_hbm, out_hbm,
          ct_v, pe1_v, oi_v, feat_v, blk_v, acc_v, sems):
        wid = lax.axis_index("s") * 2 + lax.axis_index("c")
        base = wid * _BPW
        pltpu.sync_copy(ct_hbm.at[pl.ds(base, _BPW)], ct_v)
        pltpu.sync_copy(pe1_hbm.at[pl.ds(base, _BPW)], pe1_v)
        pltpu.sync_copy(oi_hbm.at[pl.ds(base * 16, _BPW * 16)], oi_v)
        pltpu.sync_copy(feat_hbm.at[pl.ds(base * _D, _BPW * _D)], feat_v)

        lane = lax.iota(jnp.int32, 16)
        zero = jnp.zeros((16,), jnp.float32)

        def extract(ref, i):
            v16 = ref[pl.ds((i >> 4) * 16, 16)]
            return jnp.sum(jnp.where(lane == (i & 15), v16, 0))

        def enqueue(e):
            cb = (e & 0x3FFF) - 1
            slot = e >> 14
            pltpu.async_copy(
                tableT_hbm.at[:, pl.ds(pl.multiple_of(cb * 128, 128), 128)],
                blk_v.at[slot],
                sems.at[slot],
            )

        def prologue(i, carry):
            e1 = extract(pe1_v, i)

            @pl.when(e1 > 0)
            def _():
                enqueue(e1)

            return carry

        lax.fori_loop(0, _NBUF - 1, prologue, 0)

        def body(i, accs):
            c = extract(ct_v, i)
            slu = (c >> 1) & 7
            e2 = c >> 4

            @pl.when(e2 > 0)
            def _():
                enqueue(e2)

            @pl.when((c & 1) == 1)
            def _():
                pltpu.make_async_copy(
                    tableT_hbm.at[:, pl.ds(0, 128)],
                    blk_v.at[slu],
                    sems.at[slu],
                ).wait()

            o16 = oi_v[pl.ds(i * 16, 16)]
            fb = i * _D
            accs4 = list(accs)
            for kk in range(4):
                jvec = kk * 16 + lane
                cv = plsc.load_gather(blk_v.at[slu], [jvec, o16])
                d = feat_v[pl.ds(fb + kk * 16, 16)] - cv
                accs4[kk] = accs4[kk] + d * d
            return tuple(accs4)

        a0, a1, a2, a3 = lax.fori_loop(0, _BPW, body, (zero, zero, zero, zero))
        acc_v[pl.ds(0, 16)] = (a0 + a1) + (a2 + a3)
        for j in range(1, 8):
            acc_v[pl.ds(j * 16, 16)] = zero
        pltpu.sync_copy(acc_v, out_hbm.at[wid])

    return k


_sc_kernel = _make_kernel()


def _schedule(y_sorted_2d):
    cb = y_sorted_2d >> 7
    prev = jnp.concatenate([cb[:, :1] - 1, cb[:, :-1]], axis=1)
    nb = (cb != prev).astype(jnp.int32)
    bidx = jnp.cumsum(nb, axis=1) - 1
    slot = bidx % _NBUF
    pk = (cb + 1) + (slot << 14)
    cols = jnp.arange(_BPW, dtype=jnp.int32)[None, :]
    pe1 = jnp.where((cols < _NBUF - 1) & (nb == 1), pk, 0)
    sh = _NBUF - 1
    enq = jnp.concatenate(
        [jnp.where(nb[:, sh:] == 1, pk[:, sh:], 0),
         jnp.zeros((_NW, sh), jnp.int32)], axis=1)
    ctrl = nb + (slot << 1) + (enq << 4)
    return ctrl, pe1


def kernel(feat, labels, centers):
    y = labels.astype(jnp.int32)
    order = jnp.argsort(y)
    ys = y[order].reshape(_NW, _BPW)
    featp = feat[order]
    ctrl, pe1 = _schedule(ys)
    oi = jnp.broadcast_to((ys & 127).reshape(-1)[:, None], (_B, 16))
    partials = _sc_kernel(
        featp.reshape(-1), ctrl.reshape(-1), pe1.reshape(-1),
        oi.reshape(-1), centers.T)
    return jnp.sum(partials) * (0.5 / _B)


# speedup vs baseline: 3.9199x; 1.0665x over previous
import functools
import jax
import jax.numpy as jnp
from jax import lax
from jax.experimental import pallas as pl
from jax.experimental.pallas import tpu as pltpu
from jax.experimental.pallas import tpu_sc as plsc

_B = 16384
_D = 64
_NW = 32
_BPW = _B // _NW
_NBUF = 8


def _make_kernel():
    mesh = plsc.VectorSubcoreMesh(core_axis_name="c", subcore_axis_name="s")

    @functools.partial(
        pl.kernel,
        mesh=mesh,
        out_type=jax.ShapeDtypeStruct((_NW, 128), jnp.float32),
        compiler_params=pltpu.CompilerParams(needs_layout_passes=False),
        scratch_types=[
            pltpu.VMEM((_BPW,), jnp.int32),
            pltpu.VMEM((_BPW,), jnp.int32),
            pltpu.VMEM((_BPW,), jnp.int32),
            pltpu.VMEM((_BPW * _D,), jnp.float32),
            pltpu.VMEM((_NBUF, _D, 128), jnp.float32),
            pltpu.VMEM((128,), jnp.float32),
            pltpu.SemaphoreType.DMA((_NBUF,)),
        ],
    )
    def k(feat_hbm, pa_hbm, pe1_hbm, pe2_hbm, tableT_hbm, out_hbm,
          pa_v, pe1_v, pe2_v, feat_v, blk_v, acc_v, sems):
        wid = lax.axis_index("s") * 2 + lax.axis_index("c")
        base = wid * _BPW
        pltpu.sync_copy(pa_hbm.at[pl.ds(base, _BPW)], pa_v)
        pltpu.sync_copy(pe1_hbm.at[pl.ds(base, _BPW)], pe1_v)
        pltpu.sync_copy(pe2_hbm.at[pl.ds(base, _BPW)], pe2_v)
        pltpu.sync_copy(feat_hbm.at[pl.ds(base * _D, _BPW * _D)], feat_v)

        lane = lax.iota(jnp.int32, 16)
        zero = jnp.zeros((16,), jnp.float32)

        def extract(ref, i):
            v16 = ref[pl.ds((i >> 4) * 16, 16)]
            return jnp.sum(jnp.where(lane == (i & 15), v16, 0))

        def enqueue(e):
            cb = (e & 0x3FFF) - 1
            slot = e >> 14
            pltpu.async_copy(
                tableT_hbm.at[:, pl.ds(pl.multiple_of(cb * 128, 128), 128)],
                blk_v.at[slot],
                sems.at[slot],
            )

        def prologue(i, carry):
            e1 = extract(pe1_v, i)

            @pl.when(e1 > 0)
            def _():
                enqueue(e1)

            return carry

        lax.fori_loop(0, _NBUF - 1, prologue, 0)

        def body(i, accs):
            a = extract(pa_v, i)
            e2 = extract(pe2_v, i)
            slu = (a >> 21) & 7

            @pl.when(e2 > 0)
            def _():
                enqueue(e2)

            @pl.when(((a >> 20) & 1) == 1)
            def _():
                pltpu.make_async_copy(
                    tableT_hbm.at[:, pl.ds(0, 128)],
                    blk_v.at[slu],
                    sems.at[slu],
                ).wait()

            o16 = jnp.broadcast_to(a & 127, (16,))
            fb = i * _D
            accs4 = list(accs)
            for kk in range(4):
                jvec = kk * 16 + lane
                cv = plsc.load_gather(blk_v.at[slu], [jvec, o16])
                d = feat_v[pl.ds(fb + kk * 16, 16)] - cv
                accs4[kk] = accs4[kk] + d * d
            return tuple(accs4)

        a0, a1, a2, a3 = lax.fori_loop(0, _BPW, body, (zero, zero, zero, zero))
        acc_v[pl.ds(0, 16)] = (a0 + a1) + (a2 + a3)
        for j in range(1, 8):
            acc_v[pl.ds(j * 16, 16)] = zero
        pltpu.sync_copy(acc_v, out_hbm.at[wid])

    return k


_sc_kernel = _make_kernel()


def _schedule(y_sorted_2d):
    cb = y_sorted_2d >> 7
    prev = jnp.concatenate([cb[:, :1] - 1, cb[:, :-1]], axis=1)
    nb = (cb != prev).astype(jnp.int32)
    bidx = jnp.cumsum(nb, axis=1) - 1
    slot = bidx % _NBUF
    pk = (cb + 1) + (slot << 14)
    cols = jnp.arange(_BPW, dtype=jnp.int32)[None, :]
    pe1 = jnp.where((cols < _NBUF - 1) & (nb == 1), pk, 0)
    sh = _NBUF - 1
    pe2 = jnp.concatenate(
        [jnp.where(nb[:, sh:] == 1, pk[:, sh:], 0),
         jnp.zeros((_NW, sh), jnp.int32)], axis=1)
    pa = y_sorted_2d + (nb << 20) + (slot << 21)
    return pa, pe1, pe2


def kernel(feat, labels, centers):
    y = labels.astype(jnp.int32)
    order = jnp.argsort(y)
    ys = y[order].reshape(_NW, _BPW)
    featp = feat[order]
    pa, pe1, pe2 = _schedule(ys)
    partials = _sc_kernel(
        featp.reshape(-1), pa.reshape(-1), pe1.reshape(-1),
        pe2.reshape(-1), centers.T)
    return jnp.sum(partials) * (0.5 / _B)
